# baseline (speedup 1.0000x reference)
import functools

import jax
import jax.numpy as jnp
from jax import lax
from jax.experimental import pallas as pl
from jax.experimental.pallas import tpu as pltpu
from jax.experimental.pallas import tpu_sc as plsc

N = 16384
C = 512
H = 16
K = 512
HD = C // H
NP = N // K
SCALE = 0.17677669529663687

_CHUNK = 64


@functools.lru_cache(maxsize=None)
def _make_sc_gather_rows():
    info = plsc.get_sparse_core_info()
    nc, ns = info.num_cores, info.num_subcores
    nw = nc * ns
    rows_per_w = N // nw
    nchunks = rows_per_w // _CHUNK
    mesh = plsc.VectorSubcoreMesh(core_axis_name="c", subcore_axis_name="s")

    nbuf = 3

    @functools.partial(
        pl.kernel,
        mesh=mesh,
        out_type=jax.ShapeDtypeStruct((N, C), jnp.float32),
        scratch_types=[
            pltpu.VMEM((rows_per_w,), jnp.int32),
            *[pltpu.VMEM((_CHUNK, C), jnp.float32) for _ in range(nbuf)],
            *[pltpu.SemaphoreType.DMA for _ in range(2 * nbuf)],
        ],
    )
    def _sc_gather_rows(table, idx, out, idx_v, *scratch):
        rows = scratch[:nbuf]
        gsem = scratch[nbuf:2 * nbuf]
        wsem = scratch[2 * nbuf:]
        wid = lax.axis_index("s") * nc + lax.axis_index("c")
        base = wid * rows_per_w
        pltpu.sync_copy(idx.at[pl.ds(base, rows_per_w)], idx_v)
        g_h = [None] * nbuf
        w_h = [None] * nbuf
        for c in range(nchunks + 1):
            if c < nchunks:
                b = c % nbuf
                if w_h[b] is not None:
                    w_h[b].wait()
                g_h[b] = pltpu.async_copy(
                    table.at[idx_v.at[pl.ds(c * _CHUNK, _CHUNK)]],
                    rows[b], gsem[b])
            if c >= 1:
                pb = (c - 1) % nbuf
                g_h[pb].wait()
                w_h[pb] = pltpu.async_copy(
                    rows[pb], out.at[pl.ds(base + (c - 1) * _CHUNK, _CHUNK)],
                    wsem[pb])
        for h in w_h:
            h.wait()

    return _sc_gather_rows


def _attn_body(x_ref, wqkv_ref, bqkv_ref, wproj_ref, bproj_ref, o_ref):
    x = x_ref[...].astype(jnp.bfloat16)
    qkv = (
        jnp.dot(x, wqkv_ref[...], preferred_element_type=jnp.float32)
        + bqkv_ref[...]
    )
    q_all = qkv[:, :C].astype(jnp.bfloat16)
    k_all = qkv[:, C:2 * C].astype(jnp.bfloat16)
    v_all = qkv[:, 2 * C:].astype(jnp.bfloat16)
    ones_blk = jnp.ones((K, HD), dtype=jnp.bfloat16)
    outs = []
    for h in range(H):
        sl = slice(h * HD, (h + 1) * HD)
        s = lax.dot_general(
            q_all[:, sl], k_all[:, sl], (((1,), (1,)), ((), ())),
            preferred_element_type=jnp.float32,
        )
        e = jnp.exp2(s).astype(jnp.bfloat16)
        v1 = jnp.concatenate([v_all[:, sl], ones_blk], axis=1)
        ov = jnp.dot(e, v1, preferred_element_type=jnp.float32)
        o_h = ov[:, :HD] * (1.0 / ov[:, HD:])
        outs.append(o_h.astype(jnp.bfloat16))
    o = jnp.concatenate(outs, axis=1)
    o_ref[...] = (
        jnp.dot(o, wproj_ref[...].astype(jnp.bfloat16),
                preferred_element_type=jnp.float32)
        + bproj_ref[...]
    )


_attn_call = pl.pallas_call(
    _attn_body,
    grid=(NP,),
    in_specs=[
        pl.BlockSpec((K, C), lambda p: (p, 0)),
        pl.BlockSpec((C, 3 * C), lambda p: (0, 0)),
        pl.BlockSpec((1, 3 * C), lambda p: (0, 0)),
        pl.BlockSpec((C, C), lambda p: (0, 0)),
        pl.BlockSpec((1, C), lambda p: (0, 0)),
    ],
    out_specs=pl.BlockSpec((K, C), lambda p: (p, 0)),
    out_shape=jax.ShapeDtypeStruct((N, C), jnp.float32),
)


def kernel(feat, offset, serialized_order, serialized_inverse,
           W_qkv, b_qkv, W_proj, b_proj):
    order = serialized_order.reshape(N).astype(jnp.int32)
    inverse = serialized_inverse.reshape(N).astype(jnp.int32)
    s_log2e = SCALE * 1.4426950408889634
    col_scale = jnp.concatenate(
        [jnp.full((C,), s_log2e, jnp.float32), jnp.ones((2 * C,), jnp.float32)]
    )
    wqkv_bf = (W_qkv * col_scale[None, :]).astype(jnp.bfloat16)
    bqkv_scaled = (b_qkv * col_scale).reshape(1, 3 * C)
    wproj_bf = W_proj.astype(jnp.bfloat16)
    sc_gather_rows = _make_sc_gather_rows()
    feat_s = sc_gather_rows(feat, order)
    attn_out = _attn_call(
        feat_s, wqkv_bf, bqkv_scaled, wproj_bf, b_proj.reshape(1, C)
    )
    return sc_gather_rows(attn_out, inverse)


# speedup vs baseline: 1.0168x; 1.0168x over previous
import functools

import jax
import jax.numpy as jnp
from jax import lax
from jax.experimental import pallas as pl
from jax.experimental.pallas import tpu as pltpu
from jax.experimental.pallas import tpu_sc as plsc

N = 16384
C = 512
H = 16
K = 512
HD = C // H
NP = N // K
SCALE = 0.17677669529663687

_CHUNK = 64


@functools.lru_cache(maxsize=None)
def _make_sc_gather_rows():
    info = plsc.get_sparse_core_info()
    nc, ns = info.num_cores, info.num_subcores
    nw = nc * ns
    rows_per_w = N // nw
    nchunks = rows_per_w // _CHUNK
    mesh = plsc.VectorSubcoreMesh(core_axis_name="c", subcore_axis_name="s")

    nbuf = 3

    @functools.partial(
        pl.kernel,
        mesh=mesh,
        out_type=jax.ShapeDtypeStruct((N, C), jnp.float32),
        scratch_types=[
            pltpu.VMEM((rows_per_w,), jnp.int32),
            *[pltpu.VMEM((_CHUNK, C), jnp.float32) for _ in range(nbuf)],
            *[pltpu.SemaphoreType.DMA for _ in range(2 * nbuf)],
        ],
    )
    def _sc_gather_rows(table, idx, out, idx_v, *scratch):
        rows = scratch[:nbuf]
        gsem = scratch[nbuf:2 * nbuf]
        wsem = scratch[2 * nbuf:]
        wid = lax.axis_index("s") * nc + lax.axis_index("c")
        base = wid * rows_per_w
        pltpu.sync_copy(idx.at[pl.ds(base, rows_per_w)], idx_v)
        g_h = [None] * nbuf
        w_h = [None] * nbuf
        for c in range(nchunks + 1):
            if c < nchunks:
                b = c % nbuf
                if w_h[b] is not None:
                    w_h[b].wait()
                g_h[b] = pltpu.async_copy(
                    table.at[idx_v.at[pl.ds(c * _CHUNK, _CHUNK)]],
                    rows[b], gsem[b])
            if c >= 1:
                pb = (c - 1) % nbuf
                g_h[pb].wait()
                w_h[pb] = pltpu.async_copy(
                    rows[pb], out.at[pl.ds(base + (c - 1) * _CHUNK, _CHUNK)],
                    wsem[pb])
        for h in w_h:
            h.wait()

    return _sc_gather_rows


def _attn_body(x_ref, wqkv_ref, bqkv_ref, wproj_ref, bproj_ref, o_ref):
    x = x_ref[...].astype(jnp.bfloat16)
    qkv = (
        jnp.dot(x, wqkv_ref[...], preferred_element_type=jnp.float32)
        + bqkv_ref[...]
    )
    q_all = qkv[:, :C].astype(jnp.bfloat16)
    k_all = qkv[:, C:2 * C].astype(jnp.bfloat16)
    v_all = qkv[:, 2 * C:].astype(jnp.bfloat16)
    ones_blk = jnp.ones((K, HD), dtype=jnp.bfloat16)
    s_st = [None] * H
    e_st = [None] * H
    outs = [None] * H
    for t in range(H + 2):
        if t < H:
            sl = slice(t * HD, (t + 1) * HD)
            s_st[t] = lax.dot_general(
                q_all[:, sl], k_all[:, sl], (((1,), (1,)), ((), ())),
                preferred_element_type=jnp.float32,
            )
        if 1 <= t <= H:
            h = t - 1
            e_st[h] = jnp.exp2(s_st[h]).astype(jnp.bfloat16)
            s_st[h] = None
        if t >= 2:
            h = t - 2
            sl = slice(h * HD, (h + 1) * HD)
            v1 = jnp.concatenate([v_all[:, sl], ones_blk], axis=1)
            ov = jnp.dot(e_st[h], v1,
                         preferred_element_type=jnp.float32)
            outs[h] = (ov[:, :HD] * (1.0 / ov[:, HD:])).astype(jnp.bfloat16)
            e_st[h] = None
    o = jnp.concatenate(outs, axis=1)
    o_ref[...] = (
        jnp.dot(o, wproj_ref[...].astype(jnp.bfloat16),
                preferred_element_type=jnp.float32)
        + bproj_ref[...]
    )


_attn_call = pl.pallas_call(
    _attn_body,
    grid=(NP,),
    in_specs=[
        pl.BlockSpec((K, C), lambda p: (p, 0)),
        pl.BlockSpec((C, 3 * C), lambda p: (0, 0)),
        pl.BlockSpec((1, 3 * C), lambda p: (0, 0)),
        pl.BlockSpec((C, C), lambda p: (0, 0)),
        pl.BlockSpec((1, C), lambda p: (0, 0)),
    ],
    out_specs=pl.BlockSpec((K, C), lambda p: (p, 0)),
    out_shape=jax.ShapeDtypeStruct((N, C), jnp.float32),
)


def kernel(feat, offset, serialized_order, serialized_inverse,
           W_qkv, b_qkv, W_proj, b_proj):
    order = serialized_order.reshape(N).astype(jnp.int32)
    inverse = serialized_inverse.reshape(N).astype(jnp.int32)
    s_log2e = SCALE * 1.4426950408889634
    col_scale = jnp.concatenate(
        [jnp.full((C,), s_log2e, jnp.float32), jnp.ones((2 * C,), jnp.float32)]
    )
    wqkv_bf = (W_qkv * col_scale[None, :]).astype(jnp.bfloat16)
    bqkv_scaled = (b_qkv * col_scale).reshape(1, 3 * C)
    wproj_bf = W_proj.astype(jnp.bfloat16)
    sc_gather_rows = _make_sc_gather_rows()
    feat_s = sc_gather_rows(feat, order)
    attn_out = _attn_call(
        feat_s, wqkv_bf, bqkv_scaled, wproj_bf, b_proj.reshape(1, C)
    )
    return sc_gather_rows(attn_out, inverse)


# speedup vs baseline: 1.0311x; 1.0141x over previous
import functools

import jax
import jax.numpy as jnp
from jax import lax
from jax.experimental import pallas as pl
from jax.experimental.pallas import tpu as pltpu
from jax.experimental.pallas import tpu_sc as plsc

N = 16384
C = 512
H = 16
K = 512
HD = C // H
NP = N // K
SCALE = 0.17677669529663687

_CHUNK = 64


@functools.lru_cache(maxsize=None)
def _make_sc_gather_rows():
    info = plsc.get_sparse_core_info()
    nc, ns = info.num_cores, info.num_subcores
    nw = nc * ns
    rows_per_w = N // nw
    nchunks = rows_per_w // _CHUNK
    mesh = plsc.VectorSubcoreMesh(core_axis_name="c", subcore_axis_name="s")

    nbuf = 3

    @functools.partial(
        pl.kernel,
        mesh=mesh,
        out_type=jax.ShapeDtypeStruct((N, C), jnp.float32),
        scratch_types=[
            pltpu.VMEM((rows_per_w,), jnp.int32),
            *[pltpu.VMEM((_CHUNK, C), jnp.float32) for _ in range(nbuf)],
            *[pltpu.SemaphoreType.DMA for _ in range(2 * nbuf)],
        ],
    )
    def _sc_gather_rows(table, idx, out, idx_v, *scratch):
        rows = scratch[:nbuf]
        gsem = scratch[nbuf:2 * nbuf]
        wsem = scratch[2 * nbuf:]
        wid = lax.axis_index("s") * nc + lax.axis_index("c")
        base = wid * rows_per_w
        pltpu.sync_copy(idx.at[pl.ds(base, rows_per_w)], idx_v)
        g_h = [None] * nbuf
        w_h = [None] * nbuf
        for c in range(nchunks + 1):
            if c < nchunks:
                b = c % nbuf
                if w_h[b] is not None:
                    w_h[b].wait()
                g_h[b] = pltpu.async_copy(
                    table.at[idx_v.at[pl.ds(c * _CHUNK, _CHUNK)]],
                    rows[b], gsem[b])
            if c >= 1:
                pb = (c - 1) % nbuf
                g_h[pb].wait()
                w_h[pb] = pltpu.async_copy(
                    rows[pb], out.at[pl.ds(base + (c - 1) * _CHUNK, _CHUNK)],
                    wsem[pb])
        for h in w_h:
            h.wait()

    return _sc_gather_rows


def _attn_body(x_ref, wqkv_ref, bqkv_ref, wproj_ref, bproj_ref, o_ref):
    x = x_ref[...].astype(jnp.bfloat16)
    xa, xb = x[:K], x[K:]
    wqkv = wqkv_ref[...]
    bqkv = bqkv_ref[...]
    wproj = wproj_ref[...]
    bproj = bproj_ref[...]
    ones_blk = jnp.ones((K, HD), dtype=jnp.bfloat16)

    def qkv_part(xp, lo):
        return (
            jnp.dot(xp, wqkv[:, lo:lo + C], preferred_element_type=jnp.float32)
            + bqkv[:, lo:lo + C]
        ).astype(jnp.bfloat16)

    def pipeline(q_all, k_all, v_all, inject):
        s_st = [None] * H
        e_st = [None] * H
        outs = [None] * H
        for t in range(H + 2):
            if t < H:
                sl = slice(t * HD, (t + 1) * HD)
                s_st[t] = lax.dot_general(
                    q_all[:, sl], k_all[:, sl], (((1,), (1,)), ((), ())),
                    preferred_element_type=jnp.float32,
                )
            if 1 <= t <= H:
                e_st[t - 1] = jnp.exp2(s_st[t - 1]).astype(jnp.bfloat16)
                s_st[t - 1] = None
            if t >= 2:
                h = t - 2
                sl = slice(h * HD, (h + 1) * HD)
                v1 = jnp.concatenate([v_all[:, sl], ones_blk], axis=1)
                ov = jnp.dot(e_st[h], v1,
                             preferred_element_type=jnp.float32)
                outs[h] = (ov[:, :HD] * (1.0 / ov[:, HD:])).astype(jnp.bfloat16)
                e_st[h] = None
            f = inject.get(t)
            if f is not None:
                f()
        return outs

    qa, ka, va = qkv_part(xa, 0), qkv_part(xa, C), qkv_part(xa, 2 * C)
    res = {}
    inj_a = {
        1: lambda: res.__setitem__('qb', qkv_part(xb, 0)),
        3: lambda: res.__setitem__('kb', qkv_part(xb, C)),
        5: lambda: res.__setitem__('vb', qkv_part(xb, 2 * C)),
    }
    outs_a = pipeline(qa, ka, va, inj_a)

    def proj_a():
        oa = jnp.concatenate(outs_a, axis=1)
        o_ref[:K, :] = (
            jnp.dot(oa, wproj, preferred_element_type=jnp.float32) + bproj
        )

    outs_b = pipeline(res['qb'], res['kb'], res['vb'], {2: proj_a})
    ob = jnp.concatenate(outs_b, axis=1)
    o_ref[K:, :] = (
        jnp.dot(ob, wproj, preferred_element_type=jnp.float32) + bproj
    )


_attn_call = pl.pallas_call(
    _attn_body,
    grid=(NP // 2,),
    in_specs=[
        pl.BlockSpec((2 * K, C), lambda p: (p, 0)),
        pl.BlockSpec((C, 3 * C), lambda p: (0, 0)),
        pl.BlockSpec((1, 3 * C), lambda p: (0, 0)),
        pl.BlockSpec((C, C), lambda p: (0, 0)),
        pl.BlockSpec((1, C), lambda p: (0, 0)),
    ],
    out_specs=pl.BlockSpec((2 * K, C), lambda p: (p, 0)),
    out_shape=jax.ShapeDtypeStruct((N, C), jnp.float32),
)


def kernel(feat, offset, serialized_order, serialized_inverse,
           W_qkv, b_qkv, W_proj, b_proj):
    order = serialized_order.reshape(N).astype(jnp.int32)
    inverse = serialized_inverse.reshape(N).astype(jnp.int32)
    s_log2e = SCALE * 1.4426950408889634
    col_scale = jnp.concatenate(
        [jnp.full((C,), s_log2e, jnp.float32), jnp.ones((2 * C,), jnp.float32)]
    )
    wqkv_bf = (W_qkv * col_scale[None, :]).astype(jnp.bfloat16)
    bqkv_scaled = (b_qkv * col_scale).reshape(1, 3 * C)
    wproj_bf = W_proj.astype(jnp.bfloat16)
    sc_gather_rows = _make_sc_gather_rows()
    feat_s = sc_gather_rows(feat, order)
    attn_out = _attn_call(
        feat_s, wqkv_bf, bqkv_scaled, wproj_bf, b_proj.reshape(1, C)
    )
    return sc_gather_rows(attn_out, inverse)
